# baseline (speedup 1.0000x reference)
import jax
import jax.numpy as jnp
from jax.experimental import pallas as pl
from jax.experimental.pallas import tpu as pltpu


def _dma_body(table_ref, out_ref, sem):
    copy = pltpu.make_async_copy(table_ref, out_ref, sem)
    copy.start()
    copy.wait()


def kernel(x, pos_table):
    seqlen = x.shape[-1]
    embed = pos_table.shape[1]
    return pl.pallas_call(
        _dma_body,
        in_specs=[pl.BlockSpec(memory_space=pltpu.MemorySpace.HBM)],
        out_specs=pl.BlockSpec(memory_space=pltpu.MemorySpace.HBM),
        out_shape=jax.ShapeDtypeStruct((seqlen, embed), pos_table.dtype),
        scratch_shapes=[pltpu.SemaphoreType.DMA],
    )(pos_table)


# speedup vs baseline: 47.1666x; 47.1666x over previous
import jax
import jax.numpy as jnp
from jax.experimental import pallas as pl
from jax.experimental.pallas import tpu as pltpu

_BLK_ROWS = 4096


def _copy_body(table_ref, out_ref):
    out_ref[...] = table_ref[...]


def kernel(x, pos_table):
    seqlen = x.shape[-1]
    embed = pos_table.shape[1]
    nblk = pl.cdiv(seqlen, _BLK_ROWS)
    return pl.pallas_call(
        _copy_body,
        grid=(nblk,),
        in_specs=[pl.BlockSpec((_BLK_ROWS, embed), lambda i: (i, 0))],
        out_specs=pl.BlockSpec((_BLK_ROWS, embed), lambda i: (i, 0)),
        out_shape=jax.ShapeDtypeStruct((seqlen, embed), pos_table.dtype),
        compiler_params=pltpu.CompilerParams(
            dimension_semantics=("parallel",),
        ),
    )(pos_table)
